# baseline (speedup 1.0000x reference)
import functools

import jax
import jax.numpy as jnp
from jax import lax
from jax.experimental import pallas as pl
from jax.experimental.pallas import tpu as pltpu
from jax.experimental.pallas import tpu_sc as plsc

B = 1024
T = 1000
F = 4 * 64 * 64

_SC_INFO = plsc.get_sparse_core_info()
_NC = 1
_NS = _SC_INFO.num_subcores
_NW = _NC * _NS
_B_PER_W = B // _NW


@functools.partial(
    pl.kernel,
    mesh=plsc.VectorSubcoreMesh(core_axis_name="c", subcore_axis_name="s", num_cores=1),
    out_type=(
        jax.ShapeDtypeStruct((B,), jnp.float32),
        jax.ShapeDtypeStruct((B,), jnp.float32),
    ),
    scratch_types=[
        pltpu.VMEM((_B_PER_W,), jnp.int32),
        pltpu.VMEM((_B_PER_W,), jnp.float32),
        pltpu.VMEM((_B_PER_W,), jnp.float32),
        pltpu.SemaphoreType.DMA,
        pltpu.SemaphoreType.DMA,
    ],
)
def _sc_gather(t_hbm, tab1_hbm, tab2_hbm, c1_hbm, c2_hbm,
               idx_v, o1_v, o2_v, sem1, sem2):
    wid = lax.axis_index("s") * _NC + lax.axis_index("c")
    base = wid * _B_PER_W
    pltpu.sync_copy(t_hbm.at[pl.ds(base, _B_PER_W)], idx_v)
    cp1 = pltpu.async_copy(tab1_hbm.at[idx_v], o1_v, sem1)
    cp2 = pltpu.async_copy(tab2_hbm.at[idx_v], o2_v, sem2)
    cp1.wait()
    cp2.wait()
    pltpu.sync_copy(o1_v, c1_hbm.at[pl.ds(base, _B_PER_W)])
    pltpu.sync_copy(o2_v, c2_hbm.at[pl.ds(base, _B_PER_W)])


_CH = 256
_NCH = F // _CH
_D = 4


def _fma_body(c1_ref, c2_ref, x_ref, n_ref, o_ref, xb, nb, ob, *sems):
    sx = sems[0:_D]
    sn = sems[_D:2 * _D]
    so = sems[2 * _D:3 * _D]

    def in_x(k, d):
        return pltpu.make_async_copy(x_ref.at[pl.ds(k * _CH, _CH), :], xb.at[d], sx[d])

    def in_n(k, d):
        return pltpu.make_async_copy(n_ref.at[pl.ds(k * _CH, _CH), :], nb.at[d], sn[d])

    def out_o(k, d):
        return pltpu.make_async_copy(ob.at[d], o_ref.at[pl.ds(k * _CH, _CH), :], so[d])

    for d in range(_D):
        in_x(d, d).start()
        in_n(d, d).start()

    c1 = c1_ref[...]
    c2 = c2_ref[...]

    def compute(k, d):
        in_x(k, d).wait()
        in_n(k, d).wait()
        ob[d, :, :] = c1 * xb[d, :, :] + c2 * nb[d, :, :]
        out_o(k, d).start()

    for d in range(_D):
        compute(d, d)
        in_x(d + _D, d).start()
        in_n(d + _D, d).start()

    def group(g, carry):
        for d in range(_D):
            k = g * _D + d
            out_o(k - _D, d).wait()
            compute(k, d)
            in_x(k + _D, d).start()
            in_n(k + _D, d).start()
        return carry

    lax.fori_loop(1, _NCH // _D - 1, group, 0)

    for d in range(_D):
        k = _NCH - _D + d
        out_o(k - _D, d).wait()
        compute(k, d)
    for d in range(_D):
        out_o(_NCH - _D + d, d).wait()


@jax.jit
def _tc_fma(c1, c2, xT, nT):
    return pl.pallas_call(
        _fma_body,
        in_specs=[
            pl.BlockSpec(memory_space=pltpu.MemorySpace.VMEM),
            pl.BlockSpec(memory_space=pltpu.MemorySpace.VMEM),
            pl.BlockSpec(memory_space=pl.ANY),
            pl.BlockSpec(memory_space=pl.ANY),
        ],
        out_specs=pl.BlockSpec(memory_space=pl.ANY),
        out_shape=jax.ShapeDtypeStruct((F, B), jnp.float32),
        scratch_shapes=(
            [pltpu.VMEM((_D, _CH, B), jnp.float32)] * 3
            + [pltpu.SemaphoreType.DMA] * (3 * _D)
        ),
    )(c1, c2, xT, nT)


def kernel(x_start, t, noise, sqrt_alphas_cumprod, sqrt_one_minus_alphas_cumprod):
    c1, c2 = _sc_gather(t, sqrt_alphas_cumprod, sqrt_one_minus_alphas_cumprod)
    xT = x_start.reshape(B, F).T
    nT = noise.reshape(B, F).T
    outT = _tc_fma(c1.reshape(1, B), c2.reshape(1, B), xT, nT)
    return outT.T.reshape(x_start.shape)


# speedup vs baseline: 1.0009x; 1.0009x over previous
import functools

import jax
import jax.numpy as jnp
from jax import lax
from jax.experimental import pallas as pl
from jax.experimental.pallas import tpu as pltpu
from jax.experimental.pallas import tpu_sc as plsc

B = 1024
T = 1000
F = 4 * 64 * 64

_SC_INFO = plsc.get_sparse_core_info()
_NC = 1
_NS = _SC_INFO.num_subcores
_NW = _NC * _NS
_B_PER_W = B // _NW


@functools.partial(
    pl.kernel,
    mesh=plsc.VectorSubcoreMesh(core_axis_name="c", subcore_axis_name="s", num_cores=1),
    out_type=jax.ShapeDtypeStruct((2, B), jnp.float32),
    scratch_types=[
        pltpu.VMEM((_B_PER_W,), jnp.int32),
        pltpu.VMEM((_B_PER_W,), jnp.float32),
        pltpu.VMEM((_B_PER_W,), jnp.float32),
        pltpu.SemaphoreType.DMA,
        pltpu.SemaphoreType.DMA,
    ],
)
def _sc_gather(t_hbm, tab1_hbm, tab2_hbm, c_hbm,
               idx_v, o1_v, o2_v, sem1, sem2):
    wid = lax.axis_index("s") * _NC + lax.axis_index("c")
    base = wid * _B_PER_W
    pltpu.sync_copy(t_hbm.at[pl.ds(base, _B_PER_W)], idx_v)
    cp1 = pltpu.async_copy(tab1_hbm.at[idx_v], o1_v, sem1)
    cp2 = pltpu.async_copy(tab2_hbm.at[idx_v], o2_v, sem2)
    cp1.wait()
    pltpu.sync_copy(o1_v, c_hbm.at[0, pl.ds(base, _B_PER_W)])
    cp2.wait()
    pltpu.sync_copy(o2_v, c_hbm.at[1, pl.ds(base, _B_PER_W)])


_CH = 256
_NCH = F // _CH
_D = 4


def _fma_body(c_ref, x_ref, n_ref, o_ref, cb, xb, nb, ob, *sems):
    sx = sems[0:_D]
    sn = sems[_D:2 * _D]
    so = sems[2 * _D:3 * _D]
    sc_sem = sems[3 * _D]

    def in_x(k, d):
        return pltpu.make_async_copy(x_ref.at[pl.ds(k * _CH, _CH), :], xb.at[d], sx[d])

    def in_n(k, d):
        return pltpu.make_async_copy(n_ref.at[pl.ds(k * _CH, _CH), :], nb.at[d], sn[d])

    def out_o(k, d):
        return pltpu.make_async_copy(ob.at[d], o_ref.at[pl.ds(k * _CH, _CH), :], so[d])

    for d in range(_D):
        in_x(d, d).start()
        in_n(d, d).start()
    ccp = pltpu.make_async_copy(c_ref, cb, sc_sem)
    ccp.start()
    ccp.wait()
    c1 = cb[0:1, :]
    c2 = cb[1:2, :]

    def compute(k, d):
        in_x(k, d).wait()
        in_n(k, d).wait()
        ob[d, :, :] = c1 * xb[d, :, :] + c2 * nb[d, :, :]
        out_o(k, d).start()

    for d in range(_D):
        compute(d, d)
        in_x(d + _D, d).start()
        in_n(d + _D, d).start()

    def group(g, carry):
        for d in range(_D):
            k = g * _D + d
            out_o(k - _D, d).wait()
            compute(k, d)
            in_x(k + _D, d).start()
            in_n(k + _D, d).start()
        return carry

    lax.fori_loop(1, _NCH // _D - 1, group, 0)

    for d in range(_D):
        k = _NCH - _D + d
        out_o(k - _D, d).wait()
        compute(k, d)
    for d in range(_D):
        out_o(_NCH - _D + d, d).wait()


@jax.jit
def _tc_fma(c, xT, nT):
    return pl.pallas_call(
        _fma_body,
        in_specs=[
            pl.BlockSpec(memory_space=pl.ANY),
            pl.BlockSpec(memory_space=pl.ANY),
            pl.BlockSpec(memory_space=pl.ANY),
        ],
        out_specs=pl.BlockSpec(memory_space=pl.ANY),
        out_shape=jax.ShapeDtypeStruct((F, B), jnp.float32),
        scratch_shapes=(
            [pltpu.VMEM((2, B), jnp.float32)]
            + [pltpu.VMEM((_D, _CH, B), jnp.float32)] * 3
            + [pltpu.SemaphoreType.DMA] * (3 * _D + 1)
        ),
    )(c, xT, nT)


def kernel(x_start, t, noise, sqrt_alphas_cumprod, sqrt_one_minus_alphas_cumprod):
    c = _sc_gather(t, sqrt_alphas_cumprod, sqrt_one_minus_alphas_cumprod)
    xT = x_start.reshape(B, F).T
    nT = noise.reshape(B, F).T
    outT = _tc_fma(c, xT, nT)
    return outT.T.reshape(x_start.shape)
